# baseline (speedup 1.0000x reference)
import functools
import jax
import jax.numpy as jnp
from jax.experimental import pallas as pl

ALPHA = 0.2
JB = 2000
EPAD = 2048
NBITS = 16


def _prologue(x_ref, w_ref, w2_ref, a_lo_ref, a_hi_ref, a2_lo_ref, wc_ref,
              y_ref, w1_ref, scol_ref, sumxw_ref):
    bf16 = jnp.bfloat16
    x = x_ref[...]
    xw = jnp.dot(x, w_ref[...], preferred_element_type=jnp.float32)
    x4 = jnp.dot(x, w2_ref[...], preferred_element_type=jnp.float32)
    sumxw_ref[...] = jnp.sum(xw, axis=0, keepdims=True)
    c0 = jnp.dot(wc_ref[...], a_lo_ref[...],
                 preferred_element_type=jnp.float32)
    pe = jnp.dot(x4, a_hi_ref[...], preferred_element_type=jnp.float32) + c0
    pe = jnp.where(pe > 0, pe, ALPHA * pe)
    w1 = jnp.exp(pe - jnp.max(pe))
    w1_ref[...] = w1.astype(bf16)
    y_ref[...] = (xw * w1).astype(bf16)
    scol_ref[...] = jnp.dot(x4, a2_lo_ref[...],
                            preferred_element_type=jnp.float32)


def _pass1(adj_ref, pmat_ref, pk_ref):
    a = adj_ref[...].astype(jnp.bfloat16)
    pk_ref[...] = jnp.dot(a, pmat_ref[...], preferred_element_type=jnp.float32)


def _unpack(pk_ref):
    vi = pk_ref[...].astype(jnp.int32)
    return jnp.concatenate(
        [((vi >> t) & 1) for t in range(NBITS)], axis=1).astype(jnp.bfloat16)


def _pass1b(pk_ref, y_ref, w1_ref, num_ref, den_ref):
    j = pl.program_id(0)
    mask = _unpack(pk_ref)
    num = jnp.dot(y_ref[...].T, mask, preferred_element_type=jnp.float32)
    den = jnp.dot(w1_ref[...].T, mask, preferred_element_type=jnp.float32)

    @pl.when(j == 0)
    def _():
        num_ref[...] = jnp.zeros_like(num_ref)
        den_ref[...] = jnp.zeros_like(den_ref)

    num_ref[...] += num
    den_ref[...] += den


def _mid(num_ref, den_ref, sumxw_ref, w3_ref, a2_hi_ref,
         edge_ref, e1r_ref, rr_ref, maxr_ref, medge_ref,
         *, n_nodes, n_edges):
    den = den_ref[...]
    mean_xw_c = sumxw_ref[...].T / n_nodes
    edge_t = jnp.where(den > 0, num_ref[...] / jnp.where(den > 0, den, 1.0),
                       mean_xw_c)
    edge_ref[...] = edge_t.T
    medge_ref[...] = (jnp.sum(edge_t[:, :n_edges], axis=1, keepdims=True).T
                      / n_edges)
    e4_t = jax.lax.dot_general(w3_ref[...], edge_t, (((0,), (0,)), ((), ())),
                               preferred_element_type=jnp.float32)
    srow = jnp.dot(a2_hi_ref[...].T, e4_t,
                   preferred_element_type=jnp.float32)
    maxr_ref[...] = jnp.max(srow, keepdims=True)
    e1r_ref[...] = jnp.exp(srow)
    rr_ref[...] = jnp.exp((ALPHA - 1.0) * srow)


def _pass2(pk_ref, scol_ref, e1r_ref, rr_ref, maxr_ref,
           edge_ref, medge_ref, out_ref):
    bf16 = jnp.bfloat16
    mask = _unpack(pk_ref)
    scol = scol_ref[...]
    zc = scol + maxr_ref[0, 0]
    b = jnp.where(zc > 0, zc, ALPHA * zc)
    c1 = jnp.exp(scol - b).astype(bf16)
    c2 = jnp.exp(ALPHA * scol - b).astype(bf16)
    p = e1r_ref[...].astype(bf16) * jnp.maximum(c1, c2 * rr_ref[...].astype(bf16))
    w = mask * p
    e = edge_ref[...].astype(bf16)
    num = jnp.dot(w, e, preferred_element_type=jnp.float32)
    den = jnp.dot(w, jnp.ones((w.shape[1], 1), bf16),
                  preferred_element_type=jnp.float32)
    node = jnp.where(den > 0, num / jnp.where(den > 0, den, 1.0),
                     medge_ref[...])
    out_ref[...] = jnp.where(node > 0, node, jnp.exp(node) - 1.0)


def kernel(x, adj, weight, weight2, weight3, word_context, a, a2):
    n_nodes, d_in = x.shape
    n_edges = adj.shape[1]
    d_out = weight.shape[1]
    f32 = jnp.float32
    bf16 = jnp.bfloat16

    a_lo, a_hi = a[:d_out], a[d_out:]
    a2_lo, a2_hi = a2[:d_out], a2[d_out:]

    y, w1, scol, sumxw = pl.pallas_call(
        _prologue,
        out_shape=[
            jax.ShapeDtypeStruct((n_nodes, d_out), bf16),
            jax.ShapeDtypeStruct((n_nodes, 1), bf16),
            jax.ShapeDtypeStruct((n_nodes, 1), f32),
            jax.ShapeDtypeStruct((1, d_out), f32),
        ],
    )(x, weight, weight2, a_lo, a_hi, a2_lo, word_context)

    erow = jnp.arange(n_edges)[:, None]
    pmat = ((erow % 128 == jnp.arange(128)[None, :])
            * (2.0 ** (erow // 128))).astype(bf16)

    grid = (n_nodes // JB,)
    pk = pl.pallas_call(
        _pass1,
        grid=grid,
        in_specs=[
            pl.BlockSpec((JB, n_edges), lambda j: (j, 0)),
            pl.BlockSpec((n_edges, 128), lambda j: (0, 0)),
        ],
        out_specs=pl.BlockSpec((JB, 128), lambda j: (j, 0)),
        out_shape=jax.ShapeDtypeStruct((n_nodes, 128), f32),
    )(adj, pmat)

    num, den = pl.pallas_call(
        _pass1b,
        grid=grid,
        in_specs=[
            pl.BlockSpec((JB, 128), lambda j: (j, 0)),
            pl.BlockSpec((JB, d_out), lambda j: (j, 0)),
            pl.BlockSpec((JB, 1), lambda j: (j, 0)),
        ],
        out_specs=[
            pl.BlockSpec((d_out, EPAD), lambda j: (0, 0)),
            pl.BlockSpec((1, EPAD), lambda j: (0, 0)),
        ],
        out_shape=[
            jax.ShapeDtypeStruct((d_out, EPAD), f32),
            jax.ShapeDtypeStruct((1, EPAD), f32),
        ],
    )(pk, y, w1)

    edge, e1r, rr, maxr, medge = pl.pallas_call(
        functools.partial(_mid, n_nodes=n_nodes, n_edges=n_edges),
        out_shape=[
            jax.ShapeDtypeStruct((EPAD, d_out), f32),
            jax.ShapeDtypeStruct((1, EPAD), f32),
            jax.ShapeDtypeStruct((1, EPAD), f32),
            jax.ShapeDtypeStruct((1, 1), f32),
            jax.ShapeDtypeStruct((1, d_out), f32),
        ],
    )(num, den, sumxw, weight3, a2_hi)

    node = pl.pallas_call(
        _pass2,
        grid=grid,
        in_specs=[
            pl.BlockSpec((JB, 128), lambda j: (j, 0)),
            pl.BlockSpec((JB, 1), lambda j: (j, 0)),
            pl.BlockSpec((1, EPAD), lambda j: (0, 0)),
            pl.BlockSpec((1, EPAD), lambda j: (0, 0)),
            pl.BlockSpec((1, 1), lambda j: (0, 0)),
            pl.BlockSpec((EPAD, d_out), lambda j: (0, 0)),
            pl.BlockSpec((1, d_out), lambda j: (0, 0)),
        ],
        out_specs=pl.BlockSpec((JB, d_out), lambda j: (j, 0)),
        out_shape=jax.ShapeDtypeStruct((n_nodes, d_out), f32),
    )(pk, scol, e1r, rr, maxr, edge, medge)

    return node


# speedup vs baseline: 1.1429x; 1.1429x over previous
import jax
import jax.numpy as jnp
from jax.experimental import pallas as pl
from jax.experimental.pallas import tpu as pltpu

ALPHA = 0.2
JB = 1000
EPAD = 2048
NBITS = 16


def _pass1(x_ref, w_ref, w2_ref, a_lo_ref, a_hi_ref, a2_lo_ref, wc_ref,
           adj_ref, pmat_ref,
           pk_ref, num_ref, den_ref, scol_ref, sumxw_ref,
           y_scr, w1_scr):
    j = pl.program_id(0)
    bf16 = jnp.bfloat16

    @pl.when(j == 0)
    def _():
        x = x_ref[...]
        xw = jnp.dot(x, w_ref[...], preferred_element_type=jnp.float32)
        x4 = jnp.dot(x, w2_ref[...], preferred_element_type=jnp.float32)
        sumxw_ref[...] = jnp.sum(xw, axis=0, keepdims=True)
        c0 = jnp.dot(wc_ref[...], a_lo_ref[...],
                     preferred_element_type=jnp.float32)
        pe = jnp.dot(x4, a_hi_ref[...],
                     preferred_element_type=jnp.float32) + c0
        pe = jnp.where(pe > 0, pe, ALPHA * pe)
        w1 = jnp.exp(pe - jnp.max(pe))
        w1_scr[...] = w1
        y_scr[...] = xw * w1
        scol_ref[...] = jnp.dot(x4, a2_lo_ref[...],
                                preferred_element_type=jnp.float32)
        num_ref[...] = jnp.zeros_like(num_ref)
        den_ref[...] = jnp.zeros_like(den_ref)

    a = adj_ref[...].astype(bf16)
    pk_ref[...] = jnp.dot(a, pmat_ref[...], preferred_element_type=jnp.float32)
    y = y_scr[pl.ds(j * JB, JB), :].astype(bf16)
    w1 = w1_scr[pl.ds(j * JB, JB), :].astype(bf16)
    num_ref[...] += jnp.dot(y.T, a, preferred_element_type=jnp.float32)
    den_ref[...] += jnp.dot(w1.T, a, preferred_element_type=jnp.float32)


def _pass2(pk_ref, scol_ref, num_ref, den_ref, sumxw_ref, w3_ref, a2_hi_ref,
           out_ref,
           edge_scr, e1r_scr, rr_scr, maxr_scr, medge_scr,
           *, n_nodes, n_edges):
    j = pl.program_id(0)
    bf16 = jnp.bfloat16

    @pl.when(j == 0)
    def _():
        den1 = den_ref[...]
        mean_xw_c = sumxw_ref[...].T / n_nodes
        edge_t = jnp.where(den1 > 0,
                           num_ref[...] / jnp.where(den1 > 0, den1, 1.0),
                           mean_xw_c)
        d = edge_t.shape[0]
        edge_scr[...] = jnp.concatenate(
            [edge_t.T, jnp.zeros((EPAD - n_edges, d), jnp.float32)],
            axis=0).astype(bf16)
        medge_scr[...] = (jnp.sum(edge_t, axis=1, keepdims=True).T / n_edges)
        e4_t = jax.lax.dot_general(w3_ref[...], edge_t,
                                   (((0,), (0,)), ((), ())),
                                   preferred_element_type=jnp.float32)
        srow = jnp.dot(a2_hi_ref[...].T, e4_t,
                       preferred_element_type=jnp.float32)
        maxr_scr[...] = jnp.max(srow, keepdims=True)
        zpad = jnp.zeros((1, EPAD - n_edges), jnp.float32)
        e1r_scr[...] = jnp.concatenate([jnp.exp(srow), zpad],
                                       axis=1).astype(bf16)
        rr_scr[...] = jnp.concatenate([jnp.exp((ALPHA - 1.0) * srow), zpad],
                                      axis=1).astype(bf16)

    vi = pk_ref[...].astype(jnp.int32)
    mask = jnp.concatenate(
        [((vi >> t) & 1) for t in range(NBITS)], axis=1).astype(bf16)
    scol = scol_ref[...]
    zc = scol + maxr_scr[0, 0]
    b = jnp.where(zc > 0, zc, ALPHA * zc)
    c1 = jnp.exp(scol - b).astype(bf16)
    c2 = jnp.exp(ALPHA * scol - b).astype(bf16)
    p = e1r_scr[...] * jnp.maximum(c1, c2 * rr_scr[...])
    w = mask * p
    num2 = jnp.dot(w, edge_scr[...], preferred_element_type=jnp.float32)
    den2 = jnp.dot(w, jnp.ones((w.shape[1], 1), bf16),
                   preferred_element_type=jnp.float32)
    node = jnp.where(den2 > 0, num2 / jnp.where(den2 > 0, den2, 1.0),
                     medge_scr[...])
    out_ref[...] = jnp.where(node > 0, node, jnp.exp(node) - 1.0)


def kernel(x, adj, weight, weight2, weight3, word_context, a, a2):
    import functools
    n_nodes, d_in = x.shape
    n_edges = adj.shape[1]
    d_out = weight.shape[1]
    f32 = jnp.float32
    bf16 = jnp.bfloat16

    a_lo, a_hi = a[:d_out], a[d_out:]
    a2_lo, a2_hi = a2[:d_out], a2[d_out:]

    erow = jnp.arange(n_edges)[:, None]
    pmat = ((erow % 128 == jnp.arange(128)[None, :])
            * (2.0 ** (erow // 128))).astype(bf16)

    grid = (n_nodes // JB,)
    full = lambda shape: pl.BlockSpec(shape, lambda j: tuple(0 for _ in shape))
    rows = lambda shape: pl.BlockSpec(shape, lambda j: (j, 0))

    pk, num, den, scol, sumxw = pl.pallas_call(
        _pass1,
        grid=grid,
        in_specs=[
            full((n_nodes, d_in)),
            full((d_in, d_out)),
            full((d_in, d_out)),
            full((d_out, 1)),
            full((d_out, 1)),
            full((d_out, 1)),
            full((1, d_out)),
            rows((JB, n_edges)),
            full((n_edges, 128)),
        ],
        out_specs=[
            rows((JB, 128)),
            full((d_out, n_edges)),
            full((1, n_edges)),
            full((n_nodes, 1)),
            full((1, d_out)),
        ],
        out_shape=[
            jax.ShapeDtypeStruct((n_nodes, 128), f32),
            jax.ShapeDtypeStruct((d_out, n_edges), f32),
            jax.ShapeDtypeStruct((1, n_edges), f32),
            jax.ShapeDtypeStruct((n_nodes, 1), f32),
            jax.ShapeDtypeStruct((1, d_out), f32),
        ],
        scratch_shapes=[
            pltpu.VMEM((n_nodes, d_out), f32),
            pltpu.VMEM((n_nodes, 1), f32),
        ],
    )(x, weight, weight2, a_lo, a_hi, a2_lo, word_context, adj, pmat)

    node = pl.pallas_call(
        functools.partial(_pass2, n_nodes=n_nodes, n_edges=n_edges),
        grid=grid,
        in_specs=[
            rows((JB, 128)),
            rows((JB, 1)),
            full((d_out, n_edges)),
            full((1, n_edges)),
            full((1, d_out)),
            full((d_out, d_out)),
            full((d_out, 1)),
        ],
        out_specs=rows((JB, d_out)),
        out_shape=jax.ShapeDtypeStruct((n_nodes, d_out), f32),
        scratch_shapes=[
            pltpu.VMEM((EPAD, d_out), bf16),
            pltpu.VMEM((1, EPAD), bf16),
            pltpu.VMEM((1, EPAD), bf16),
            pltpu.VMEM((1, 1), f32),
            pltpu.VMEM((1, d_out), f32),
        ],
    )(pk, scol, num, den, sumxw, weight3, a2_hi)

    return node


# speedup vs baseline: 1.1495x; 1.0058x over previous
import jax
import jax.numpy as jnp
from jax.experimental import pallas as pl
from jax.experimental.pallas import tpu as pltpu

ALPHA = 0.2
JB = 1000
EPAD = 2048
NBITS = 16


def _pass1(x_ref, w_ref, w2_ref, a_lo_ref, a_hi_ref, a2_lo_ref, wc_ref,
           adj_ref, pmat_ref,
           pk_ref, nd_ref, scol_ref, sumxw_ref,
           ya_scr):
    j = pl.program_id(0)
    bf16 = jnp.bfloat16

    @pl.when(j == 0)
    def _():
        x = x_ref[...]
        xw = jnp.dot(x, w_ref[...], preferred_element_type=jnp.float32)
        x4 = jnp.dot(x, w2_ref[...], preferred_element_type=jnp.float32)
        sumxw_ref[...] = jnp.sum(xw, axis=0, keepdims=True)
        c0 = jnp.dot(wc_ref[...], a_lo_ref[...],
                     preferred_element_type=jnp.float32)
        pe = jnp.dot(x4, a_hi_ref[...],
                     preferred_element_type=jnp.float32) + c0
        pe = jnp.where(pe > 0, pe, ALPHA * pe)
        w1 = jnp.exp(pe - jnp.max(pe))
        n = x.shape[0]
        ya_scr[...] = jnp.concatenate(
            [xw * w1, w1, jnp.zeros((n, 7), jnp.float32)], axis=1)
        scol_ref[...] = jnp.dot(x4, a2_lo_ref[...],
                                preferred_element_type=jnp.float32)
        nd_ref[...] = jnp.zeros_like(nd_ref)

    a = adj_ref[...]
    pk_ref[...] = jnp.dot(a, pmat_ref[...], preferred_element_type=jnp.float32)
    ya = ya_scr[pl.ds(j * JB, JB), :]
    nd_ref[...] += jnp.dot(ya.T, a, preferred_element_type=jnp.float32)


def _pass2(pk_ref, scol_ref, nd_ref, sumxw_ref, w3_ref, a2_hi_ref,
           out_ref,
           edge_scr, e1r_scr, rr_scr, maxr_scr, medge_scr,
           *, n_nodes, n_edges):
    j = pl.program_id(0)
    bf16 = jnp.bfloat16

    @pl.when(j == 0)
    def _():
        den1 = nd_ref[128:129, :]
        mean_xw_c = sumxw_ref[...].T / n_nodes
        edge_t = jnp.where(den1 > 0,
                           nd_ref[0:128, :] / jnp.where(den1 > 0, den1, 1.0),
                           mean_xw_c)
        d = edge_t.shape[0]
        edge_scr[...] = jnp.concatenate(
            [edge_t.T, jnp.zeros((EPAD - n_edges, d), jnp.float32)],
            axis=0).astype(bf16)
        medge_scr[...] = (jnp.sum(edge_t, axis=1, keepdims=True).T / n_edges)
        e4_t = jax.lax.dot_general(w3_ref[...], edge_t,
                                   (((0,), (0,)), ((), ())),
                                   preferred_element_type=jnp.float32)
        srow = jnp.dot(a2_hi_ref[...].T, e4_t,
                       preferred_element_type=jnp.float32)
        maxr_scr[...] = jnp.max(srow, keepdims=True)
        zpad = jnp.zeros((1, EPAD - n_edges), jnp.float32)
        e1r_scr[...] = jnp.concatenate([jnp.exp(srow), zpad],
                                       axis=1).astype(bf16)
        rr_scr[...] = jnp.concatenate([jnp.exp((ALPHA - 1.0) * srow), zpad],
                                      axis=1).astype(bf16)

    vi = pk_ref[...].astype(jnp.int32)
    mask = jnp.concatenate(
        [((vi >> t) & 1) for t in range(NBITS)], axis=1).astype(bf16)
    scol = scol_ref[...]
    zc = scol + maxr_scr[0, 0]
    b = jnp.where(zc > 0, zc, ALPHA * zc)
    c1 = jnp.exp(scol - b).astype(bf16)
    c2 = jnp.exp(ALPHA * scol - b).astype(bf16)
    p = e1r_scr[...] * jnp.maximum(c1, c2 * rr_scr[...])
    w = mask * p
    num2 = jnp.dot(w, edge_scr[...], preferred_element_type=jnp.float32)
    den2 = jnp.dot(w, jnp.ones((w.shape[1], 1), bf16),
                   preferred_element_type=jnp.float32)
    node = jnp.where(den2 > 0, num2 / jnp.where(den2 > 0, den2, 1.0),
                     medge_scr[...])
    out_ref[...] = jnp.where(node > 0, node, jnp.exp(node) - 1.0)


def kernel(x, adj, weight, weight2, weight3, word_context, a, a2):
    import functools
    n_nodes, d_in = x.shape
    n_edges = adj.shape[1]
    d_out = weight.shape[1]
    f32 = jnp.float32
    bf16 = jnp.bfloat16

    a_lo, a_hi = a[:d_out], a[d_out:]
    a2_lo, a2_hi = a2[:d_out], a2[d_out:]

    erow = jnp.arange(n_edges)[:, None]
    pmat = ((erow % 128 == jnp.arange(128)[None, :])
            * (2.0 ** (erow // 128))).astype(f32)

    grid = (n_nodes // JB,)
    full = lambda shape: pl.BlockSpec(shape, lambda j: tuple(0 for _ in shape))
    rows = lambda shape: pl.BlockSpec(shape, lambda j: (j, 0))

    pk, nd, scol, sumxw = pl.pallas_call(
        _pass1,
        grid=grid,
        in_specs=[
            full((n_nodes, d_in)),
            full((d_in, d_out)),
            full((d_in, d_out)),
            full((d_out, 1)),
            full((d_out, 1)),
            full((d_out, 1)),
            full((1, d_out)),
            rows((JB, n_edges)),
            full((n_edges, 128)),
        ],
        out_specs=[
            rows((JB, 128)),
            full((136, n_edges)),
            full((n_nodes, 1)),
            full((1, d_out)),
        ],
        out_shape=[
            jax.ShapeDtypeStruct((n_nodes, 128), f32),
            jax.ShapeDtypeStruct((136, n_edges), f32),
            jax.ShapeDtypeStruct((n_nodes, 1), f32),
            jax.ShapeDtypeStruct((1, d_out), f32),
        ],
        scratch_shapes=[
            pltpu.VMEM((n_nodes, 136), f32),
        ],
    )(x, weight, weight2, a_lo, a_hi, a2_lo, word_context, adj, pmat)

    node = pl.pallas_call(
        functools.partial(_pass2, n_nodes=n_nodes, n_edges=n_edges),
        grid=grid,
        in_specs=[
            rows((JB, 128)),
            rows((JB, 1)),
            full((136, n_edges)),
            full((1, d_out)),
            full((d_out, d_out)),
            full((d_out, 1)),
        ],
        out_specs=rows((JB, d_out)),
        out_shape=jax.ShapeDtypeStruct((n_nodes, d_out), f32),
        scratch_shapes=[
            pltpu.VMEM((EPAD, d_out), bf16),
            pltpu.VMEM((1, EPAD), bf16),
            pltpu.VMEM((1, EPAD), bf16),
            pltpu.VMEM((1, 1), f32),
            pltpu.VMEM((1, d_out), f32),
        ],
    )(pk, scol, nd, sumxw, weight3, a2_hi)

    return node
